# initial kernel scaffold (unmeasured)
import jax
import jax.numpy as jnp
from jax import lax
from jax.experimental import pallas as pl
from jax.experimental.pallas import tpu as pltpu

N_DEV = 4
SQ = 2048
D = 1024
HEADS = 8
DH = 128
W = 128
BQ = 256
WIN = 512
SCALE = 0.08838834764831843


def kernel(x, Wq, K_ext, V_ext, Wo):
    def body(x_ref, wq_ref, wo_ref, k_hbm, v_hbm, out_ref,
             comm, xb, q_ref, k_buf, v_buf,
             send_sems, recv_sems, k_sem, v_sem):
        my = lax.axis_index("i")
        left = lax.rem(my + N_DEV - 1, N_DEV)
        right = lax.rem(my + 1, N_DEV)

        barrier = pltpu.get_barrier_semaphore()
        for nbr in (left, right):
            pl.semaphore_signal(barrier, inc=1, device_id=(nbr,),
                                device_id_type=pl.DeviceIdType.MESH)
        pl.semaphore_wait(barrier, 2)

        xb[...] = x_ref[0].astype(jnp.bfloat16)
        comm[0, :D, :] = wq_ref[...].astype(jnp.bfloat16)
        comm[0, D:, :] = wo_ref[...].astype(jnp.bfloat16)

        for h in range(N_DEV):
            slot = h % 2
            g = lax.rem(my - h + N_DEV, N_DEV)

            if h < N_DEV - 1:
                rdma = pltpu.make_async_remote_copy(
                    src_ref=comm.at[slot],
                    dst_ref=comm.at[1 - slot],
                    send_sem=send_sems.at[slot],
                    recv_sem=recv_sems.at[1 - slot],
                    device_id=(right,),
                    device_id_type=pl.DeviceIdType.MESH,
                )
                rdma.start()

            g8 = g * HEADS
            copies = []
            for hh in range(HEADS):
                copies.append(pltpu.make_async_copy(
                    k_hbm.at[my, :, g8 + hh, :], k_buf.at[hh], k_sem))
                copies.append(pltpu.make_async_copy(
                    v_hbm.at[my, :, g8 + hh, :], v_buf.at[hh], v_sem))
            for c in copies:
                c.start()

            q_ref[...] = lax.dot_general(
                xb[...], comm[slot, :D, :],
                (((1,), (0,)), ((), ())),
                preferred_element_type=jnp.bfloat16)

            for c in copies:
                c.wait()

            def blk(t, carry):
                s = jnp.clip(t * BQ - W, 0, SQ - WIN)
                rows = pl.ds(t * BQ, BQ)
                q_idx = t * BQ + lax.broadcasted_iota(jnp.int32, (BQ, WIN), 0)
                k_idx = s + lax.broadcasted_iota(jnp.int32, (BQ, WIN), 1)
                mask = jnp.abs(q_idx - k_idx) <= W
                acc = jnp.zeros((BQ, D), jnp.float32)
                for hh in range(HEADS):
                    qh = q_ref[rows, pl.ds(hh * DH, DH)]
                    kh = k_buf[hh, pl.ds(s, WIN), :].astype(jnp.bfloat16)
                    vh = v_buf[hh, pl.ds(s, WIN), :].astype(jnp.bfloat16)
                    sc = lax.dot_general(
                        qh, kh, (((1,), (1,)), ((), ())),
                        preferred_element_type=jnp.float32) * SCALE
                    sc = jnp.where(mask, sc, -1e9)
                    m = jnp.max(sc, axis=1, keepdims=True)
                    wgt = jnp.exp(sc - m)
                    wgt = wgt / jnp.sum(wgt, axis=1, keepdims=True)
                    ctx = lax.dot_general(
                        wgt.astype(jnp.bfloat16), vh,
                        (((1,), (0,)), ((), ())),
                        preferred_element_type=jnp.float32)
                    woh = comm[slot, pl.ds(D + hh * DH, DH), :]
                    acc = acc + lax.dot_general(
                        ctx.astype(jnp.bfloat16), woh,
                        (((1,), (0,)), ((), ())),
                        preferred_element_type=jnp.float32)
                if h == 0:
                    out_ref[0, rows, :] = acc
                else:
                    out_ref[0, rows, :] = out_ref[0, rows, :] + acc
                return carry

            lax.fori_loop(0, SQ // BQ, blk, 0)

            if h < N_DEV - 1:
                rdma.wait()

    return pl.pallas_call(
        body,
        out_shape=jax.ShapeDtypeStruct((1, SQ, D), jnp.float32),
        in_specs=[
            pl.BlockSpec(memory_space=pltpu.VMEM),
            pl.BlockSpec(memory_space=pltpu.VMEM),
            pl.BlockSpec(memory_space=pltpu.VMEM),
            pl.BlockSpec(memory_space=pltpu.ANY),
            pl.BlockSpec(memory_space=pltpu.ANY),
        ],
        out_specs=pl.BlockSpec(memory_space=pltpu.VMEM),
        scratch_shapes=[
            pltpu.VMEM((2, 2 * D, D), jnp.bfloat16),
            pltpu.VMEM((SQ, D), jnp.bfloat16),
            pltpu.VMEM((SQ, D), jnp.bfloat16),
            pltpu.VMEM((HEADS, SQ, DH), jnp.float32),
            pltpu.VMEM((HEADS, SQ, DH), jnp.float32),
            pltpu.SemaphoreType.DMA((2,)),
            pltpu.SemaphoreType.DMA((2,)),
            pltpu.SemaphoreType.DMA,
            pltpu.SemaphoreType.DMA,
        ],
        compiler_params=pltpu.CompilerParams(collective_id=0),
    )(x, Wq, Wo, K_ext, V_ext)


# baseline (device time: 217496 ns/iter reference)
import jax
import jax.numpy as jnp
from jax import lax
from jax.experimental import pallas as pl
from jax.experimental.pallas import tpu as pltpu

N_DEV = 4
SQ = 2048
D = 1024
HEADS = 8
DH = 128
W = 128
BQ = 256
WIN = 512
SCALE = 0.08838834764831843


def kernel(x, Wq, K_ext, V_ext, Wo):
    def body(x_ref, wq_ref, wo_ref, k_hbm, v_hbm, out_ref,
             comm, q_ref, k_buf, v_buf,
             send_sems, recv_sems, k_sem, v_sem):
        my = lax.axis_index("i")
        left = lax.rem(my + N_DEV - 1, N_DEV)
        right = lax.rem(my + 1, N_DEV)

        barrier = pltpu.get_barrier_semaphore()
        for nbr in (left, right):
            pl.semaphore_signal(barrier, inc=1, device_id=(nbr,),
                                device_id_type=pl.DeviceIdType.MESH)
        pl.semaphore_wait(barrier, 2)

        comm[0, :D, :] = wq_ref[...].astype(jnp.bfloat16)
        comm[0, D:, :] = wo_ref[...].astype(jnp.bfloat16)

        for h in range(N_DEV):
            slot = h % 2
            g = lax.rem(my - h + N_DEV, N_DEV)

            if h < N_DEV - 1:
                rdma = pltpu.make_async_remote_copy(
                    src_ref=comm.at[slot],
                    dst_ref=comm.at[1 - slot],
                    send_sem=send_sems.at[slot],
                    recv_sem=recv_sems.at[1 - slot],
                    device_id=(right,),
                    device_id_type=pl.DeviceIdType.MESH,
                )
                rdma.start()

            g8 = g * HEADS
            copies = []
            for hh in range(HEADS):
                copies.append(pltpu.make_async_copy(
                    k_hbm.at[my, :, g8 + hh, :], k_buf.at[hh], k_sem))
                copies.append(pltpu.make_async_copy(
                    v_hbm.at[my, :, g8 + hh, :], v_buf.at[hh], v_sem))
            for c in copies:
                c.start()

            def qproj(t, carry):
                rows = pl.ds(t * BQ, BQ)
                q_ref[rows, :] = lax.dot_general(
                    x_ref[0, rows, :].astype(jnp.bfloat16),
                    comm[slot, :D, :],
                    (((1,), (0,)), ((), ())),
                    preferred_element_type=jnp.float32).astype(jnp.bfloat16)
                return carry

            lax.fori_loop(0, SQ // BQ, qproj, 0)

            for c in copies:
                c.wait()

            def blk(t, carry):
                s = jnp.clip(t * BQ - W, 0, SQ - WIN)
                rows = pl.ds(t * BQ, BQ)
                q_idx = t * BQ + lax.broadcasted_iota(jnp.int32, (BQ, WIN), 0)
                k_idx = s + lax.broadcasted_iota(jnp.int32, (BQ, WIN), 1)
                mask = jnp.abs(q_idx - k_idx) <= W
                acc = jnp.zeros((BQ, D), jnp.float32)
                for hh in range(HEADS):
                    qh = q_ref[rows, pl.ds(hh * DH, DH)]
                    kh = k_buf[hh, pl.ds(s, WIN), :].astype(jnp.bfloat16)
                    vh = v_buf[hh, pl.ds(s, WIN), :].astype(jnp.bfloat16)
                    sc = lax.dot_general(
                        qh, kh, (((1,), (1,)), ((), ())),
                        preferred_element_type=jnp.float32) * SCALE
                    sc = jnp.where(mask, sc, -1e9)
                    m = jnp.max(sc, axis=1, keepdims=True)
                    wgt = jnp.exp(sc - m)
                    wgt = wgt / jnp.sum(wgt, axis=1, keepdims=True)
                    ctx = lax.dot_general(
                        wgt.astype(jnp.bfloat16), vh,
                        (((1,), (0,)), ((), ())),
                        preferred_element_type=jnp.float32)
                    woh = comm[slot, pl.ds(D + hh * DH, DH), :]
                    acc = acc + lax.dot_general(
                        ctx.astype(jnp.bfloat16), woh,
                        (((1,), (0,)), ((), ())),
                        preferred_element_type=jnp.float32)
                if h == 0:
                    out_ref[0, rows, :] = acc
                else:
                    out_ref[0, rows, :] = out_ref[0, rows, :] + acc
                return carry

            lax.fori_loop(0, SQ // BQ, blk, 0)

            if h < N_DEV - 1:
                rdma.wait()

    return pl.pallas_call(
        body,
        out_shape=jax.ShapeDtypeStruct((1, SQ, D), jnp.float32),
        in_specs=[
            pl.BlockSpec(memory_space=pltpu.MemorySpace.VMEM),
            pl.BlockSpec(memory_space=pltpu.MemorySpace.VMEM),
            pl.BlockSpec(memory_space=pltpu.MemorySpace.VMEM),
            pl.BlockSpec(memory_space=pl.ANY),
            pl.BlockSpec(memory_space=pl.ANY),
        ],
        out_specs=pl.BlockSpec(memory_space=pltpu.MemorySpace.VMEM),
        scratch_shapes=[
            pltpu.VMEM((2, 2 * D, D), jnp.bfloat16),
            pltpu.VMEM((SQ, D), jnp.bfloat16),
            pltpu.VMEM((HEADS, SQ, DH), jnp.float32),
            pltpu.VMEM((HEADS, SQ, DH), jnp.float32),
            pltpu.SemaphoreType.DMA((2,)),
            pltpu.SemaphoreType.DMA((2,)),
            pltpu.SemaphoreType.DMA,
            pltpu.SemaphoreType.DMA,
        ],
        compiler_params=pltpu.CompilerParams(
            collective_id=0, vmem_limit_bytes=110 * 1024 * 1024),
    )(x, Wq, Wo, K_ext, V_ext)


# device time: 216876 ns/iter; 1.0029x vs baseline; 1.0029x over previous
import jax
import jax.numpy as jnp
from jax import lax
from jax.experimental import pallas as pl
from jax.experimental.pallas import tpu as pltpu

N_DEV = 4
SQ = 2048
D = 1024
HEADS = 8
HH = HEADS // 2
DH = 128
HD = HH * DH
W = 128
BQ = 256
WIN = 512
SCALE = 0.08838834764831843


def kernel(x, Wq, K_ext, V_ext, Wo):
    def body(x_ref, wq_ref, wo_ref, k_hbm, v_hbm, out_ref,
             wq_r, wo_r, wq_l, wo_l, q_ref, k_buf, v_buf,
             send_sems, recv_sems, k_sem, v_sem):
        my = lax.axis_index("i")
        left = lax.rem(my + N_DEV - 1, N_DEV)
        right = lax.rem(my + 1, N_DEV)

        wq_r[0] = wq_ref[:, :HD].astype(jnp.bfloat16)
        wq_l[0] = wq_ref[:, HD:].astype(jnp.bfloat16)
        wo_r[0] = wo_ref[:HD, :].astype(jnp.bfloat16)
        wo_l[0] = wo_ref[HD:, :].astype(jnp.bfloat16)

        barrier = pltpu.get_barrier_semaphore()
        for nbr in (left, right):
            pl.semaphore_signal(barrier, inc=1, device_id=(nbr,),
                                device_id_type=pl.DeviceIdType.MESH)
        pl.semaphore_wait(barrier, 2)

        for h in range(N_DEV):
            slot = h % 2
            g_r = lax.rem(my - h + N_DEV, N_DEV)
            g_l = lax.rem(my + h, N_DEV)

            rdmas = []
            if h < N_DEV - 1:
                for idx, (buf, dest) in enumerate(
                        ((wq_r, right), (wo_r, right),
                         (wq_l, left), (wo_l, left))):
                    rdma = pltpu.make_async_remote_copy(
                        src_ref=buf.at[slot],
                        dst_ref=buf.at[1 - slot],
                        send_sem=send_sems.at[idx, slot],
                        recv_sem=recv_sems.at[idx, 1 - slot],
                        device_id=(dest,),
                        device_id_type=pl.DeviceIdType.MESH,
                    )
                    rdma.start()
                    rdmas.append(rdma)

            copies = []
            for hh in range(HEADS):
                src_head = jnp.where(hh < HH, g_r * HEADS + hh,
                                     g_l * HEADS + hh)
                copies.append(pltpu.make_async_copy(
                    k_hbm.at[my, :, src_head, :], k_buf.at[hh], k_sem))
                copies.append(pltpu.make_async_copy(
                    v_hbm.at[my, :, src_head, :], v_buf.at[hh], v_sem))
            for c in copies:
                c.start()

            def qproj(t, carry):
                rows = pl.ds(t * BQ, BQ)
                xb = x_ref[0, rows, :].astype(jnp.bfloat16)
                q_ref[rows, :HD] = lax.dot_general(
                    xb, wq_r[slot],
                    (((1,), (0,)), ((), ())),
                    preferred_element_type=jnp.float32).astype(jnp.bfloat16)
                q_ref[rows, HD:] = lax.dot_general(
                    xb, wq_l[slot],
                    (((1,), (0,)), ((), ())),
                    preferred_element_type=jnp.float32).astype(jnp.bfloat16)
                return carry

            lax.fori_loop(0, SQ // BQ, qproj, 0)

            for c in copies:
                c.wait()

            def blk(t, carry):
                s = jnp.clip(t * BQ - W, 0, SQ - WIN)
                rows = pl.ds(t * BQ, BQ)
                q_idx = t * BQ + lax.broadcasted_iota(jnp.int32, (BQ, WIN), 0)
                k_idx = s + lax.broadcasted_iota(jnp.int32, (BQ, WIN), 1)
                mask = jnp.abs(q_idx - k_idx) <= W
                acc = jnp.zeros((BQ, D), jnp.float32)
                for hh in range(HEADS):
                    qh = q_ref[rows, pl.ds(hh * DH, DH)]
                    kh = k_buf[hh, pl.ds(s, WIN), :].astype(jnp.bfloat16)
                    vh = v_buf[hh, pl.ds(s, WIN), :].astype(jnp.bfloat16)
                    sc = lax.dot_general(
                        qh, kh, (((1,), (1,)), ((), ())),
                        preferred_element_type=jnp.float32) * SCALE
                    sc = jnp.where(mask, sc, -1e9)
                    m = jnp.max(sc, axis=1, keepdims=True)
                    wgt = jnp.exp(sc - m)
                    wgt = wgt / jnp.sum(wgt, axis=1, keepdims=True)
                    ctx = lax.dot_general(
                        wgt.astype(jnp.bfloat16), vh,
                        (((1,), (0,)), ((), ())),
                        preferred_element_type=jnp.float32)
                    if hh < HH:
                        woh = wo_r[slot, pl.ds(hh * DH, DH), :]
                    else:
                        woh = wo_l[slot, pl.ds((hh - HH) * DH, DH), :]
                    acc = acc + lax.dot_general(
                        ctx.astype(jnp.bfloat16), woh,
                        (((1,), (0,)), ((), ())),
                        preferred_element_type=jnp.float32)
                if h == 0:
                    out_ref[0, rows, :] = acc
                else:
                    out_ref[0, rows, :] = out_ref[0, rows, :] + acc
                return carry

            lax.fori_loop(0, SQ // BQ, blk, 0)

            for rdma in rdmas:
                rdma.wait()

    return pl.pallas_call(
        body,
        out_shape=jax.ShapeDtypeStruct((1, SQ, D), jnp.float32),
        in_specs=[
            pl.BlockSpec(memory_space=pltpu.MemorySpace.VMEM),
            pl.BlockSpec(memory_space=pltpu.MemorySpace.VMEM),
            pl.BlockSpec(memory_space=pltpu.MemorySpace.VMEM),
            pl.BlockSpec(memory_space=pl.ANY),
            pl.BlockSpec(memory_space=pl.ANY),
        ],
        out_specs=pl.BlockSpec(memory_space=pltpu.MemorySpace.VMEM),
        scratch_shapes=[
            pltpu.VMEM((2, D, HD), jnp.bfloat16),
            pltpu.VMEM((2, HD, D), jnp.bfloat16),
            pltpu.VMEM((2, D, HD), jnp.bfloat16),
            pltpu.VMEM((2, HD, D), jnp.bfloat16),
            pltpu.VMEM((SQ, D), jnp.bfloat16),
            pltpu.VMEM((HEADS, SQ, DH), jnp.float32),
            pltpu.VMEM((HEADS, SQ, DH), jnp.float32),
            pltpu.SemaphoreType.DMA((4, 2)),
            pltpu.SemaphoreType.DMA((4, 2)),
            pltpu.SemaphoreType.DMA,
            pltpu.SemaphoreType.DMA,
        ],
        compiler_params=pltpu.CompilerParams(
            collective_id=0, vmem_limit_bytes=110 * 1024 * 1024),
    )(x, Wq, Wo, K_ext, V_ext)


# device time: 122855 ns/iter; 1.7703x vs baseline; 1.7653x over previous
import jax
import jax.numpy as jnp
from jax import lax
from jax.experimental import pallas as pl
from jax.experimental.pallas import tpu as pltpu

N_DEV = 4
SQ = 2048
D = 1024
HEADS = 8
HH = HEADS // 2
DH = 128
HD = HH * DH
W = 128
QB = 512
BQ = 512
WIN = 768
SCALE = 0.08838834764831843


def kernel(x, Wq, K_ext, V_ext, Wo):
    def body(x_ref, wq_ref, wo_ref, k_hbm, v_hbm, out_ref,
             wq_r, wo_r, wq_l, wo_l, wo_full, xb_ref, q_ref, k_buf, v_buf,
             send_sems, recv_sems, k_sem, v_sem):
        my = lax.axis_index("i")
        left = lax.rem(my + N_DEV - 1, N_DEV)
        right = lax.rem(my + 1, N_DEV)

        wq_r[0] = (wq_ref[:, :HD] * SCALE).astype(jnp.bfloat16)
        wq_l[0] = (wq_ref[:, HD:] * SCALE).astype(jnp.bfloat16)
        wo_r[0] = wo_ref[:HD, :].astype(jnp.bfloat16)
        wo_l[0] = wo_ref[HD:, :].astype(jnp.bfloat16)
        xb_ref[...] = x_ref[0].astype(jnp.bfloat16)

        barrier = pltpu.get_barrier_semaphore()
        for nbr in (left, right):
            pl.semaphore_signal(barrier, inc=1, device_id=(nbr,),
                                device_id_type=pl.DeviceIdType.MESH)
        pl.semaphore_wait(barrier, 2)

        for h in range(N_DEV):
            slot = h % 2
            g_r = lax.rem(my - h + N_DEV, N_DEV)
            g_l = lax.rem(my + h, N_DEV)

            rdmas = []
            if h < N_DEV - 1:
                for idx, (buf, dest) in enumerate(
                        ((wq_r, right), (wo_r, right),
                         (wq_l, left), (wo_l, left))):
                    rdma = pltpu.make_async_remote_copy(
                        src_ref=buf.at[slot],
                        dst_ref=buf.at[1 - slot],
                        send_sem=send_sems.at[idx, slot],
                        recv_sem=recv_sems.at[idx, 1 - slot],
                        device_id=(dest,),
                        device_id_type=pl.DeviceIdType.MESH,
                    )
                    rdma.start()
                    rdmas.append(rdma)

            copies = []
            for hh in range(HEADS):
                src_head = jnp.where(hh < HH, g_r * HEADS + hh,
                                     g_l * HEADS + hh)
                copies.append(pltpu.make_async_copy(
                    k_hbm.at[my, :, src_head, :], k_buf.at[hh], k_sem))
                copies.append(pltpu.make_async_copy(
                    v_hbm.at[my, :, src_head, :], v_buf.at[hh], v_sem))
            for c in copies:
                c.start()

            def qproj(t, carry):
                rows = pl.ds(t * QB, QB)
                xb = xb_ref[rows, :]
                q_ref[rows, :HD] = lax.dot_general(
                    xb, wq_r[slot],
                    (((1,), (0,)), ((), ())),
                    preferred_element_type=jnp.float32).astype(jnp.bfloat16)
                q_ref[rows, HD:] = lax.dot_general(
                    xb, wq_l[slot],
                    (((1,), (0,)), ((), ())),
                    preferred_element_type=jnp.float32).astype(jnp.bfloat16)
                return carry

            lax.fori_loop(0, SQ // QB, qproj, 0)

            wo_full[:HD, :] = wo_r[slot]
            wo_full[HD:, :] = wo_l[slot]

            for c in copies:
                c.wait()

            def blk(t, carry):
                s = jnp.clip(t * (BQ // 128) - 1, 0, (SQ - WIN) // 128) * 128
                rows = pl.ds(t * BQ, BQ)
                q_idx = t * BQ + lax.broadcasted_iota(jnp.int32, (BQ, WIN), 0)
                k_idx = s + lax.broadcasted_iota(jnp.int32, (BQ, WIN), 1)
                bias = jnp.where(jnp.abs(q_idx - k_idx) <= W,
                                 jnp.float32(0), jnp.float32(-1e9))
                ctxs = []
                for hh in range(HEADS):
                    qh = q_ref[rows, pl.ds(hh * DH, DH)]
                    kh = k_buf[hh, pl.ds(s, WIN), :].astype(jnp.bfloat16)
                    vh = v_buf[hh, pl.ds(s, WIN), :].astype(jnp.bfloat16)
                    sc = lax.dot_general(
                        qh, kh, (((1,), (1,)), ((), ())),
                        preferred_element_type=jnp.float32)
                    e = jnp.exp(sc + bias)
                    denom = jnp.sum(e, axis=1, keepdims=True)
                    ctx = lax.dot_general(
                        e.astype(jnp.bfloat16), vh,
                        (((1,), (0,)), ((), ())),
                        preferred_element_type=jnp.float32)
                    ctxs.append((ctx / denom).astype(jnp.bfloat16))
                ctx_full = jnp.concatenate(ctxs, axis=1)
                acc = lax.dot_general(
                    ctx_full, wo_full[...],
                    (((1,), (0,)), ((), ())),
                    preferred_element_type=jnp.float32)
                if h == 0:
                    out_ref[0, rows, :] = acc
                else:
                    out_ref[0, rows, :] = out_ref[0, rows, :] + acc
                return carry

            lax.fori_loop(0, SQ // BQ, blk, 0)

            for rdma in rdmas:
                rdma.wait()

    return pl.pallas_call(
        body,
        out_shape=jax.ShapeDtypeStruct((1, SQ, D), jnp.float32),
        in_specs=[
            pl.BlockSpec(memory_space=pltpu.MemorySpace.VMEM),
            pl.BlockSpec(memory_space=pltpu.MemorySpace.VMEM),
            pl.BlockSpec(memory_space=pltpu.MemorySpace.VMEM),
            pl.BlockSpec(memory_space=pl.ANY),
            pl.BlockSpec(memory_space=pl.ANY),
        ],
        out_specs=pl.BlockSpec(memory_space=pltpu.MemorySpace.VMEM),
        scratch_shapes=[
            pltpu.VMEM((2, D, HD), jnp.bfloat16),
            pltpu.VMEM((2, HD, D), jnp.bfloat16),
            pltpu.VMEM((2, D, HD), jnp.bfloat16),
            pltpu.VMEM((2, HD, D), jnp.bfloat16),
            pltpu.VMEM((D, D), jnp.bfloat16),
            pltpu.VMEM((SQ, D), jnp.bfloat16),
            pltpu.VMEM((SQ, D), jnp.bfloat16),
            pltpu.VMEM((HEADS, SQ, DH), jnp.float32),
            pltpu.VMEM((HEADS, SQ, DH), jnp.float32),
            pltpu.SemaphoreType.DMA((4, 2)),
            pltpu.SemaphoreType.DMA((4, 2)),
            pltpu.SemaphoreType.DMA,
            pltpu.SemaphoreType.DMA,
        ],
        compiler_params=pltpu.CompilerParams(
            collective_id=0, vmem_limit_bytes=110 * 1024 * 1024),
    )(x, Wq, Wo, K_ext, V_ext)
